# baseline (speedup 1.0000x reference)
import functools

import jax
import jax.numpy as jnp
from jax import lax
from jax.experimental import pallas as pl
from jax.experimental.pallas import tpu as pltpu
from jax.experimental.pallas import tpu_sc as plsc

LEN = 128
NW = 32
KSLICES = 4


def _gather_hslice(x, h0, hn, tag):
    T, C, H, W = x.shape
    chunks = LEN * C
    per_w = chunks // NW
    a, b = 2 * (T - 1), 2 * (LEN - 1)

    mesh = plsc.VectorSubcoreMesh(core_axis_name="c", subcore_axis_name="s")

    @functools.partial(
        pl.kernel,
        out_type=jax.ShapeDtypeStruct((LEN, C, hn, W), x.dtype),
        mesh=mesh,
        scratch_types=[
            pltpu.VMEM((2, hn, W), x.dtype),
            pltpu.SemaphoreType.DMA,
            pltpu.SemaphoreType.DMA,
            pltpu.SemaphoreType.DMA,
            pltpu.SemaphoreType.DMA,
        ],
        name=f"sc_gather_{tag}",
    )
    def k(x_hbm, out_hbm, buf, si0, si1, so0, so1):
        wid = lax.axis_index("s") * 2 + lax.axis_index("c")
        base = wid * per_w
        sin = (si0, si1)
        sout = (so0, so1)

        def start_in(q, slot):
            o = base + q
            frame = o // C
            ch = o % C
            src = (frame * a + (LEN - 1)) // b
            return pltpu.async_copy(
                x_hbm.at[src, ch, pl.ds(h0, hn)], buf.at[slot], sin[slot]
            )

        def start_out(q, slot):
            o = base + q
            return pltpu.async_copy(
                buf.at[slot], out_hbm.at[o // C, o % C], sout[slot]
            )

        in_cp = [None, None]
        out_cp = [None, None]
        in_cp[0] = start_in(0, 0)
        for q in range(per_w):
            slot = q % 2
            nxt = (q + 1) % 2
            if q + 1 < per_w:
                if q >= 1:
                    out_cp[nxt].wait()
                in_cp[nxt] = start_in(q + 1, nxt)
            in_cp[slot].wait()
            out_cp[slot] = start_out(q, slot)
        out_cp[0].wait()
        out_cp[1].wait()

    return k(x)


def kernel(x):
    T, C, H, W = x.shape
    hn = H // KSLICES
    parts = [_gather_hslice(x, k * hn, hn, f"h{k}") for k in range(KSLICES)]
    out = jnp.zeros((LEN, C, H, W), x.dtype)
    for k in range(KSLICES):
        out = lax.dynamic_update_slice(out, parts[k], (0, 0, k * hn, 0))
    return out


# speedup vs baseline: 1.2370x; 1.2370x over previous
import numpy as np
import jax
import jax.numpy as jnp
from jax.experimental import pallas as pl
from jax.experimental.pallas import tpu as pltpu

LEN = 128


def _copy_body(x_ref, o_ref):
    o_ref[...] = x_ref[...]


def kernel(x):
    T, C, H, W = x.shape
    a, b = 2 * (T - 1), LEN - 1

    return pl.pallas_call(
        _copy_body,
        grid=(LEN,),
        in_specs=[pl.BlockSpec((1, C, H, W), lambda i: ((i * a + b) // (2 * b), 0, 0, 0))],
        out_specs=pl.BlockSpec((1, C, H, W), lambda i: (i, 0, 0, 0)),
        out_shape=jax.ShapeDtypeStruct((LEN, C, H, W), x.dtype),
        compiler_params=pltpu.CompilerParams(
            dimension_semantics=("arbitrary",),
        ),
    )(x)
